# Initial kernel scaffold; baseline (speedup 1.0000x reference)
import jax
import jax.numpy as jnp
from jax.experimental import pallas as pl


def kernel(x, table, bias):
    raise NotImplementedError("write your pallas kernel here")


# speedup vs baseline: 1.2586x; 1.2586x over previous
import functools

import jax
import jax.numpy as jnp
from jax import lax
from jax.experimental import pallas as pl
from jax.experimental.pallas import tpu as pltpu
from jax.experimental.pallas import tpu_sc as plsc

BATCH = 16384
NUM_FIELDS = 26
NUM_WORKERS = 32
ROWS_PER_W = BATCH // NUM_WORKERS
IDX_PER_W = ROWS_PER_W * NUM_FIELDS


@functools.partial(
    pl.kernel,
    out_type=jax.ShapeDtypeStruct((BATCH,), jnp.float32),
    mesh=plsc.VectorSubcoreMesh(core_axis_name="c", subcore_axis_name="s"),
    scratch_types=[
        pltpu.VMEM((IDX_PER_W,), jnp.int32),
        pltpu.VMEM((IDX_PER_W,), jnp.float32),
        pltpu.VMEM((ROWS_PER_W,), jnp.float32),
        pltpu.SemaphoreType.DMA,
    ],
)
def _emb_sum(x_hbm, table_hbm, out_hbm, idx_v, vals_v, out_v, sem):
    wid = lax.axis_index("s") * 2 + lax.axis_index("c")

    pltpu.sync_copy(x_hbm.at[wid], idx_v)
    pltpu.async_copy(table_hbm.at[idx_v], vals_v, sem).wait()

    def accum(i, _):
        def fbody(f, acc):
            return acc + vals_v[pl.ds(f * ROWS_PER_W + i * 16, 16)]

        acc = lax.fori_loop(0, NUM_FIELDS, fbody, jnp.zeros((16,), jnp.float32))
        out_v[pl.ds(i * 16, 16)] = acc
        return 0

    lax.fori_loop(0, ROWS_PER_W // 16, accum, 0)
    pltpu.sync_copy(out_v, out_hbm.at[pl.ds(wid * ROWS_PER_W, ROWS_PER_W)])


def kernel(x, table, bias):
    xw = (
        x.T.reshape(NUM_FIELDS, NUM_WORKERS, ROWS_PER_W)
        .transpose(1, 0, 2)
        .reshape(NUM_WORKERS, IDX_PER_W)
    )
    out = _emb_sum(xw, table.reshape(-1))
    return out.reshape(BATCH, 1) + bias
